# Initial kernel scaffold; baseline (speedup 1.0000x reference)
import functools

import jax
import jax.numpy as jnp
from jax import lax
from jax.experimental import pallas as pl
from jax.experimental.pallas import tpu as pltpu
from jax.experimental.pallas import tpu_sc as plsc

N = 10000
IN = 128
H = 64
C = 32

NPAD = 10240
CH = 128
NC = 2
NS = 16
NW = NC * NS
ROWS_PER_TILE = NPAD // NS


def _make_sc_agg(nchunks: int, with_deg: bool):
    out_type = [jax.ShapeDtypeStruct((NC, NPAD, H), jnp.float32)]
    if with_deg:
        out_type.append(jax.ShapeDtypeStruct((NC, NPAD), jnp.float32))

    scratch = [
        pltpu.VMEM((nchunks, CH), jnp.int32),
        pltpu.VMEM((nchunks, CH), jnp.int32),
        pltpu.VMEM((CH, H), jnp.float32),
        pltpu.VMEM((CH,), jnp.float32),
        pltpu.VMEM_SHARED((NPAD, H), jnp.float32),
        pltpu.VMEM_SHARED((NPAD,), jnp.float32),
        pltpu.SemaphoreType.DMA,
    ]

    mesh = plsc.VectorSubcoreMesh(core_axis_name="c", subcore_axis_name="s")

    def body(table, src, dst, z64, z1, *rest):
        if with_deg:
            out, degout, sidx, didx, rows, ones, acc, dacc, sem = rest
        else:
            out, sidx, didx, rows, ones, acc, dacc, sem = rest
            degout = None
        c = lax.axis_index("c")
        s = lax.axis_index("s")
        tid = c * NS + s

        for i in range(CH // 16):
            ones[pl.ds(i * 16, 16)] = jnp.ones((16,), jnp.float32)

        @pl.when(s == 0)
        def _():
            pltpu.sync_copy(z64, acc)
            pltpu.sync_copy(z1, dacc)

        base = tid * (nchunks * CH)
        pltpu.sync_copy(src.at[pl.ds(base, nchunks * CH)], sidx)
        pltpu.sync_copy(dst.at[pl.ds(base, nchunks * CH)], didx)
        plsc.subcore_barrier()

        def step(j, carry):
            pltpu.async_copy(table.at[sidx.at[j]], rows, sem).wait()
            pltpu.sync_copy(rows, acc.at[didx.at[j]], add=True)
            if with_deg:
                pltpu.sync_copy(ones, dacc.at[didx.at[j]], add=True)
            return carry

        lax.fori_loop(0, nchunks, step, 0)
        plsc.subcore_barrier()

        r0 = s * ROWS_PER_TILE
        pltpu.sync_copy(acc.at[pl.ds(r0, ROWS_PER_TILE)],
                        out.at[c, pl.ds(r0, ROWS_PER_TILE)])
        if with_deg:
            pltpu.sync_copy(dacc.at[pl.ds(r0, ROWS_PER_TILE)],
                            degout.at[c, pl.ds(r0, ROWS_PER_TILE)])

    return pl.kernel(body, out_type=out_type, mesh=mesh, scratch_types=scratch)


BN = 2000


def _tc1_body(x_ref, wl_ref, wr_ref, y_ref, r_ref):
    x = x_ref[...]
    y_ref[...] = jnp.dot(x, wl_ref[...], preferred_element_type=jnp.float32)
    r_ref[...] = jnp.dot(x, wr_ref[...], preferred_element_type=jnp.float32)


def _tc2_body(p_ref, dpt_ref, r1_ref, b1_ref, wl_ref, wr_ref, y_ref, r_ref):
    agg = p_ref[0] + p_ref[1]
    deg = dpt_ref[:, 0] + dpt_ref[:, 1]
    inv = 1.0 / jnp.maximum(deg, 1.0)
    h = jnp.maximum(agg * inv[:, None] + r1_ref[...] + b1_ref[...][None, :], 0.0)
    y_ref[...] = jnp.dot(h, wl_ref[...], preferred_element_type=jnp.float32)
    r_ref[...] = jnp.dot(h, wr_ref[...], preferred_element_type=jnp.float32)


def _tc3_body(p_ref, dpt_ref, r2_ref, b2_ref, wc_ref, bc_ref, log_ref, emb_ref):
    agg = p_ref[0] + p_ref[1]
    deg = dpt_ref[:, 0] + dpt_ref[:, 1]
    inv = 1.0 / jnp.maximum(deg, 1.0)
    emb = agg * inv[:, None] + r2_ref[...] + b2_ref[...][None, :]
    emb_ref[...] = emb
    log_ref[...] = (jnp.dot(emb, wc_ref[...], preferred_element_type=jnp.float32)
                    + bc_ref[...][None, :])


def _row_spec(d):
    return pl.BlockSpec((BN, d), lambda i: (i, 0))


def _full_spec(shape):
    return pl.BlockSpec(shape, lambda i: tuple(0 for _ in shape))


_tc1 = pl.pallas_call(
    _tc1_body,
    grid=(N // BN,),
    in_specs=[_row_spec(IN), _full_spec((IN, H)), _full_spec((IN, H))],
    out_specs=[_row_spec(H), _row_spec(H)],
    out_shape=[jax.ShapeDtypeStruct((N, H), jnp.float32)] * 2,
)

_tc2 = pl.pallas_call(
    _tc2_body,
    grid=(N // BN,),
    in_specs=[
        pl.BlockSpec((NC, BN, H), lambda i: (0, i, 0)),
        _row_spec(NC),
        _row_spec(H),
        _full_spec((H,)),
        _full_spec((H, H)),
        _full_spec((H, H)),
    ],
    out_specs=[_row_spec(H), _row_spec(H)],
    out_shape=[jax.ShapeDtypeStruct((N, H), jnp.float32)] * 2,
)

_tc3 = pl.pallas_call(
    _tc3_body,
    grid=(N // BN,),
    in_specs=[
        pl.BlockSpec((NC, BN, H), lambda i: (0, i, 0)),
        _row_spec(NC),
        _row_spec(H),
        _full_spec((H,)),
        _full_spec((H, C)),
        _full_spec((C,)),
    ],
    out_specs=[_row_spec(C), _row_spec(H)],
    out_shape=[jax.ShapeDtypeStruct((N, C), jnp.float32),
               jax.ShapeDtypeStruct((N, H), jnp.float32)],
)


def kernel(x, edge_index, W1l, b1, W1r, W2l, b2, W2r, Wc, bc):
    E = edge_index.shape[1]
    epw = -(-E // (NW * CH)) * CH
    nchunks = epw // CH
    e_pad = NW * epw

    src = edge_index[0]
    dst = edge_index[1]
    pad = e_pad - E
    if pad:
        src = jnp.concatenate([src, jnp.zeros((pad,), jnp.int32)])
        dst = jnp.concatenate([dst, jnp.full((pad,), NPAD - 1, jnp.int32)])

    z64 = jnp.zeros((NPAD, H), jnp.float32)
    z1 = jnp.zeros((NPAD,), jnp.float32)

    sc_agg1 = _make_sc_agg(nchunks, with_deg=True)
    sc_agg2 = _make_sc_agg(nchunks, with_deg=False)

    y1, r1 = _tc1(x, W1l, W1r)
    p1, dp = sc_agg1(y1, src, dst, z64, z1)
    dpt = dp.T[:N]
    y2, r2 = _tc2(p1[:, :N], dpt, r1, b1, W2l, W2r)
    (p2,) = sc_agg2(y2, src, dst, z64, z1)
    logits, emb = _tc3(p2[:, :N], dpt, r2, b2, Wc, bc)
    return logits, emb


# speedup vs baseline: 6.9013x; 6.9013x over previous
import functools

import jax
import jax.numpy as jnp
from jax import lax
from jax.experimental import pallas as pl
from jax.experimental.pallas import tpu as pltpu
from jax.experimental.pallas import tpu_sc as plsc

N = 10000
IN = 128
H = 64
C = 32

NPAD = 10240
CH = 128
NC = 2
NS = 16
NW = NC * NS
ROWS_PER_TILE = NPAD // NS


def _make_sc_agg(nchunks: int, with_deg: bool):
    out_type = [jax.ShapeDtypeStruct((NC, NPAD, H), jnp.float32)]
    if with_deg:
        out_type.append(jax.ShapeDtypeStruct((NC, NPAD), jnp.float32))

    scratch = [
        pltpu.VMEM((nchunks, CH), jnp.int32),
        pltpu.VMEM((nchunks, CH), jnp.int32),
        pltpu.VMEM((CH, H), jnp.float32),
        pltpu.VMEM((CH,), jnp.float32),
        pltpu.VMEM_SHARED((NPAD, H), jnp.float32),
        pltpu.VMEM_SHARED((NPAD,), jnp.float32),
        pltpu.SemaphoreType.DMA,
    ]

    mesh = plsc.VectorSubcoreMesh(core_axis_name="c", subcore_axis_name="s",
                                  num_cores=NC, num_subcores=NS)

    def body(table, src, dst, z64, z1, *rest):
        if with_deg:
            out, degout, sidx, didx, rows, ones, acc, dacc, sem = rest
        else:
            out, sidx, didx, rows, ones, acc, dacc, sem = rest
            degout = None
        c = lax.axis_index("c")
        s = lax.axis_index("s")
        tid = c * NS + s

        for i in range(CH // 16):
            ones[pl.ds(i * 16, 16)] = jnp.ones((16,), jnp.float32)

        @pl.when(s == 0)
        def _():
            pltpu.sync_copy(z64, acc)
            pltpu.sync_copy(z1, dacc)

        pltpu.sync_copy(src.at[tid], sidx)
        pltpu.sync_copy(dst.at[tid], didx)
        plsc.subcore_barrier()

        def step(j, carry):
            pltpu.async_copy(table.at[sidx.at[j]], rows, sem).wait()
            pltpu.sync_copy(rows, acc.at[didx.at[j]], add=True)
            if with_deg:
                pltpu.sync_copy(ones, dacc.at[didx.at[j]], add=True)
            return carry

        lax.fori_loop(0, nchunks, step, 0)
        plsc.subcore_barrier()

        r0 = s * ROWS_PER_TILE
        pltpu.sync_copy(acc.at[pl.ds(r0, ROWS_PER_TILE)],
                        out.at[c, pl.ds(r0, ROWS_PER_TILE)])
        if with_deg:
            pltpu.sync_copy(dacc.at[pl.ds(r0, ROWS_PER_TILE)],
                            degout.at[c, pl.ds(r0, ROWS_PER_TILE)])

    return pl.kernel(body, out_type=out_type, mesh=mesh, scratch_types=scratch,
                     compiler_params=pltpu.CompilerParams(use_tc_tiling_on_sc=False))


BN = 2000


def _tc1_body(x_ref, wl_ref, wr_ref, y_ref, r_ref):
    x = x_ref[...]
    y_ref[...] = jnp.dot(x, wl_ref[...], preferred_element_type=jnp.float32)
    r_ref[...] = jnp.dot(x, wr_ref[...], preferred_element_type=jnp.float32)


def _tc2_body(p_ref, dpt_ref, r1_ref, b1_ref, wl_ref, wr_ref, y_ref, r_ref):
    agg = p_ref[0] + p_ref[1]
    deg = dpt_ref[:, 0] + dpt_ref[:, 1]
    inv = 1.0 / jnp.maximum(deg, 1.0)
    h = jnp.maximum(agg * inv[:, None] + r1_ref[...] + b1_ref[...][None, :], 0.0)
    y_ref[...] = jnp.dot(h, wl_ref[...], preferred_element_type=jnp.float32)
    r_ref[...] = jnp.dot(h, wr_ref[...], preferred_element_type=jnp.float32)


def _tc3_body(p_ref, dpt_ref, r2_ref, b2_ref, wc_ref, bc_ref, log_ref, emb_ref):
    agg = p_ref[0] + p_ref[1]
    deg = dpt_ref[:, 0] + dpt_ref[:, 1]
    inv = 1.0 / jnp.maximum(deg, 1.0)
    emb = agg * inv[:, None] + r2_ref[...] + b2_ref[...][None, :]
    emb_ref[...] = emb
    log_ref[...] = (jnp.dot(emb, wc_ref[...], preferred_element_type=jnp.float32)
                    + bc_ref[...][None, :])


def _row_spec(d):
    return pl.BlockSpec((BN, d), lambda i: (i, 0))


def _full_spec(shape):
    return pl.BlockSpec(shape, lambda i: tuple(0 for _ in shape))


_tc1 = pl.pallas_call(
    _tc1_body,
    grid=(N // BN,),
    in_specs=[_row_spec(IN), _full_spec((IN, H)), _full_spec((IN, H))],
    out_specs=[_row_spec(H), _row_spec(H)],
    out_shape=[jax.ShapeDtypeStruct((N, H), jnp.float32)] * 2,
)

_tc2 = pl.pallas_call(
    _tc2_body,
    grid=(N // BN,),
    in_specs=[
        pl.BlockSpec((NC, BN, H), lambda i: (0, i, 0)),
        _row_spec(NC),
        _row_spec(H),
        _full_spec((H,)),
        _full_spec((H, H)),
        _full_spec((H, H)),
    ],
    out_specs=[_row_spec(H), _row_spec(H)],
    out_shape=[jax.ShapeDtypeStruct((N, H), jnp.float32)] * 2,
)

_tc3 = pl.pallas_call(
    _tc3_body,
    grid=(N // BN,),
    in_specs=[
        pl.BlockSpec((NC, BN, H), lambda i: (0, i, 0)),
        _row_spec(NC),
        _row_spec(H),
        _full_spec((H,)),
        _full_spec((H, C)),
        _full_spec((C,)),
    ],
    out_specs=[_row_spec(C), _row_spec(H)],
    out_shape=[jax.ShapeDtypeStruct((N, C), jnp.float32),
               jax.ShapeDtypeStruct((N, H), jnp.float32)],
)


def kernel(x, edge_index, W1l, b1, W1r, W2l, b2, W2r, Wc, bc):
    E = edge_index.shape[1]
    epw = -(-E // (NW * CH)) * CH
    nchunks = epw // CH
    e_pad = NW * epw

    src = edge_index[0]
    dst = edge_index[1]
    pad = e_pad - E
    if pad:
        src = jnp.concatenate([src, jnp.zeros((pad,), jnp.int32)])
        dst = jnp.concatenate([dst, jnp.full((pad,), NPAD - 1, jnp.int32)])
    src = src.reshape(NW, nchunks, CH)
    dst = dst.reshape(NW, nchunks, CH)

    z64 = jnp.zeros((NPAD, H), jnp.float32)
    z1 = jnp.zeros((NPAD,), jnp.float32)

    sc_agg1 = _make_sc_agg(nchunks, with_deg=True)
    sc_agg2 = _make_sc_agg(nchunks, with_deg=False)

    y1, r1 = _tc1(x, W1l, W1r)
    p1, dp = sc_agg1(y1, src, dst, z64, z1)
    dpt = dp.T
    y2, r2 = _tc2(p1, dpt, r1, b1, W2l, W2r)
    p2 = sc_agg2(y2, src, dst, z64, z1)
    p2 = p2[0] if isinstance(p2, (list, tuple)) else p2
    logits, emb = _tc3(p2, dpt, r2, b2, Wc, bc)
    return logits, emb
